# baseline (speedup 1.0000x reference)
import functools

import jax
import jax.numpy as jnp
import numpy as np
from jax import lax
from jax.experimental import pallas as pl
from jax.experimental.pallas import tpu as pltpu
from jax.experimental.pallas import tpu_sc as plsc

VOCAB = 1000
VPAD = 1024
EMB = 128
VEC = 64
B = 16384
L = 200

NC = 2
NS = 16
LN = 16
NW = NC * NS
R = 32
CV = R * VPAD
CS = R * (VPAD // 2)
BH = B // 2

_mesh = plsc.VectorSubcoreMesh(core_axis_name="c", subcore_axis_name="s")


def _make_sc_counts(nrows, row_off):
    rows_per_w = nrows // NW
    pairs_per_w = rows_per_w // 2
    nchunk = rows_per_w // R

    @functools.partial(
        pl.kernel,
        out_type=jax.ShapeDtypeStruct((nrows * (VPAD // 2),), jnp.float32),
        mesh=_mesh,
        scratch_types=[
            pltpu.VMEM((R, L), jnp.int32),
            pltpu.VMEM((R, L), jnp.int32),
            pltpu.VMEM((CV,), jnp.float32),
            pltpu.VMEM((CV,), jnp.float32),
            pltpu.VMEM((CS,), jnp.float32),
            pltpu.VMEM((CS,), jnp.float32),
            pltpu.SemaphoreType.DMA,
            pltpu.SemaphoreType.DMA,
        ],
        compiler_params=pltpu.CompilerParams(needs_layout_passes=False),
    )
    def sc_counts(x_hbm, out_hbm, idx0, idx1, cnt0, cnt1, st0, st1,
                  sem0, sem1):
        wid = lax.axis_index("s") * NC + lax.axis_index("c")
        ones = jnp.ones((LN,), jnp.float32)
        zeros = jnp.zeros((LN,), jnp.float32)
        lane = lax.iota(jnp.int32, LN)
        tail_mask = lane >= 8
        idx_b = (idx0, idx1)
        cnt_b = (cnt0, cnt1)
        st_b = (st0, st1)
        sem_b = (sem0, sem1)

        ZU = 8

        def zbody(i, carry):
            for u in range(ZU):
                cnt0[pl.ds((i * ZU + u) * LN, LN)] = zeros
                cnt1[pl.ds((i * ZU + u) * LN, LN)] = zeros
            return carry

        lax.fori_loop(0, CV // (LN * ZU), zbody, 0)

        def swizzle(xv, roff):
            return ((xv & 1) << 9) + lax.shift_right_logical(xv, 1) + roff

        bias_idx0 = lane * VPAD + (VOCAB >> 1)
        bias_idx1 = bias_idx0 + LN * VPAD
        for cbuf in (cnt0, cnt1):
            plsc.store_scatter(cbuf, [bias_idx0], ones)
            plsc.store_scatter(cbuf, [bias_idx1], ones)

        RPI = 2

        def rows_pass(slot, r0, value_vec, add):
            xs, masks = [], []
            for rr in range(RPI):
                r = r0 + rr
                roff = r * VPAD
                for j in range(13):
                    if j < 12:
                        xv = idx_b[slot][r, pl.ds(j * LN, LN)]
                        masks.append(None)
                    else:
                        xv = idx_b[slot][r, pl.ds(L - LN, LN)]
                        masks.append(tail_mask)
                    xs.append(swizzle(xv, roff))
            for fidx, mask in zip(xs, masks):
                if add:
                    plsc.addupdate_scatter(cnt_b[slot], [fidx], value_vec,
                                           mask=mask)
                else:
                    plsc.store_scatter(cnt_b[slot], [fidx], zeros, mask=mask)

        def add_pass(slot):
            def body(i, carry):
                rows_pass(slot, i * RPI, ones, True)
                return carry
            lax.fori_loop(0, R // RPI, body, 0)

        def zero_pass(slot):
            def body(i, carry):
                rows_pass(slot, i * RPI, zeros, False)
                return carry
            lax.fori_loop(0, R // RPI, body, 0)

        def pack_pass(slot):
            def body(i, carry):
                lr = lax.shift_right_logical(i, 1)
                half_col = (i & 1) * (VPAD // 4)
                cbase = lr * VPAD + half_col
                sbase = ((lr & 15) * VPAD + lax.shift_right_logical(lr, 4)
                         * (VPAD // 2) + half_col)
                avs = [cnt_b[slot][pl.ds(cbase + g * LN, LN)]
                       for g in range(16)]
                bvs = [cnt_b[slot][pl.ds(cbase + (VPAD // 2) + g * LN, LN)]
                       for g in range(16)]
                for g in range(16):
                    packed = plsc.pack(avs[g], bvs[g],
                                       format=plsc.PackFormat.INTERLEAVED)
                    st_b[slot][pl.ds(sbase + g * LN, LN)] = plsc.bitcast(
                        packed, jnp.float32)
                return carry
            lax.fori_loop(0, 2 * R, body, 0)

        def out_copy(slot, g):
            pair0 = wid * pairs_per_w + g * (R // 2)
            return pltpu.make_async_copy(
                st_b[slot], out_hbm.at[pl.ds(pair0 * VPAD, CS)], sem_b[slot])

        def process(slot, g):
            pair0 = wid * pairs_per_w + g * (R // 2)
            blk = pair0 // (BH // 16)
            q0 = pair0 % (BH // 16)
            base0 = row_off + blk * (2 * (BH // 16)) + q0
            pltpu.sync_copy(x_hbm.at[pl.ds(base0, R // 2)],
                            idx_b[slot].at[pl.ds(0, R // 2)])
            pltpu.sync_copy(x_hbm.at[pl.ds(base0 + BH // 16, R // 2)],
                            idx_b[slot].at[pl.ds(R // 2, R // 2)])
            add_pass(slot)
            pack_pass(slot)
            out_copy(slot, g).start()
            zero_pass(slot)

        process(0, 0)
        process(1, 1)

        def chunk_body(g, carry):
            def for_slot(slot):
                @pl.when(lax.rem(g, 2) == slot)
                def _():
                    out_copy(slot, g - 2).wait()
                    process(slot, g)
            for_slot(0)
            for_slot(1)
            return carry

        lax.fori_loop(2, nchunk, chunk_body, 0)
        out_copy(0, nchunk - 2).wait()
        out_copy(1, nchunk - 1).wait()

    return sc_counts


_sc_counts_a = _make_sc_counts(BH, 0)
_sc_counts_b = _make_sc_counts(BH, BH)

_TAU = np.empty((VPAD,), np.int32)
for _p in range(VPAD):
    _s, _r = divmod(_p, 256)
    _h, _m = divmod(_r, 128)
    _TAU[_p] = _s * 256 + 2 * _m + _h
_BIN_ROW = int(np.where(_TAU == VOCAB)[0][0])


def _prep_body(emb_ref, win_ref, bin_ref, t_ref):
    t_ref[...] = lax.dot_general(
        emb_ref[...], win_ref[...], (((1,), (1,)), ((), ())),
        preferred_element_type=jnp.float32)
    t_ref[pl.ds(_BIN_ROW, 1), :] = bin_ref[...]


BM = 1024
BM2 = BM // 2
NBH = BH // BM


def _compute_block(cnt_ref, t_ref, wout_ref, bout_ref, out_ref):
    halves = []
    for half in range(2):
        acc = None
        for s in range(4):
            wi = lax.bitcast_convert_type(cnt_ref[:, half * 4 + s, :],
                                          jnp.int32)
            lo = lax.bitcast_convert_type(wi << 16, jnp.float32)
            hi = lax.bitcast_convert_type(wi & jnp.int32(-65536), jnp.float32)
            part = (jnp.dot(lo, t_ref[pl.ds(s * 256, 128), :],
                            preferred_element_type=jnp.float32)
                    + jnp.dot(hi, t_ref[pl.ds(s * 256 + 128, 128), :],
                              preferred_element_type=jnp.float32))
            acc = part if acc is None else acc + part
        halves.append(acc)
    h = jnp.concatenate(halves, axis=0)
    e0 = (lax.broadcasted_iota(jnp.int32, (BM, VEC), 1) == 0).astype(jnp.float32)
    out_ref[...] = (
        lax.dot_general(wout_ref[...], h, (((1,), (1,)), ((), ())),
                        preferred_element_type=jnp.float32)
        + lax.dot_general(bout_ref[...], e0, (((1,), (1,)), ((), ())),
                          preferred_element_type=jnp.float32))


def _main_a_body(cnt_ref, t_ref, wout_ref, bout_ref, out_ref):
    _compute_block(cnt_ref, t_ref, wout_ref, bout_ref, out_ref)


def _main_b_body(alias_ref, cnt_ref, t_ref, wout_ref, bout_ref, out_ref):
    del alias_ref
    _compute_block(cnt_ref, t_ref, wout_ref, bout_ref, out_ref)


_weight_specs = [
    pl.BlockSpec((VPAD, VEC), lambda i: (0, 0)),
    pl.BlockSpec((VOCAB, VEC), lambda i: (0, 0)),
    pl.BlockSpec((VOCAB, VEC), lambda i: (0, 0)),
]


def kernel(x, embeddings, W_in, b_in, W_out, b_out):
    counts_a = _sc_counts_a(x)
    counts_b = _sc_counts_b(x)
    emb_perm = jnp.pad(embeddings, ((0, VPAD - VOCAB), (0, 0)))[_TAU]
    t = pl.pallas_call(
        _prep_body,
        out_shape=jax.ShapeDtypeStruct((VPAD, VEC), jnp.float32),
    )(emb_perm, W_in, b_in.reshape(1, VEC))
    bout_col = jnp.pad(b_out.reshape(VOCAB, 1), ((0, 0), (0, VEC - 1)))
    out_shape = jax.ShapeDtypeStruct((VOCAB, B), jnp.float32)
    out_a = pl.pallas_call(
        _main_a_body,
        grid=(NBH,),
        in_specs=[pl.BlockSpec((BM2, 8, 128), lambda i: (i, 0, 0))]
        + _weight_specs,
        out_specs=pl.BlockSpec((VOCAB, BM), lambda i: (0, i)),
        out_shape=out_shape,
    )(counts_a.reshape(BH // 2, 8, 128), t, W_out, bout_col)
    out_t = pl.pallas_call(
        _main_b_body,
        grid=(NBH,),
        in_specs=[pl.BlockSpec(memory_space=pltpu.MemorySpace.HBM)]
        + [pl.BlockSpec((BM2, 8, 128), lambda i: (i, 0, 0))]
        + _weight_specs,
        out_specs=pl.BlockSpec((VOCAB, BM), lambda i: (0, i + NBH)),
        out_shape=out_shape,
        input_output_aliases={0: 0},
    )(out_a, counts_b.reshape(BH // 2, 8, 128), t, W_out, bout_col)
    return out_t.T


# speedup vs baseline: 1.3800x; 1.3800x over previous
import functools

import jax
import jax.numpy as jnp
from jax import lax
from jax.experimental import pallas as pl
from jax.experimental.pallas import tpu as pltpu
from jax.experimental.pallas import tpu_sc as plsc

VOCAB = 1000
VPAD = 1024
EMB = 128
VEC = 64
B = 16384
L = 200

NC = 2
NS = 16
LN = 16
NW = NC * NS
R = 32
CV = R * VPAD
BH = B // 2

_mesh = plsc.VectorSubcoreMesh(core_axis_name="c", subcore_axis_name="s")


def _make_sc_counts(nrows, row_off):
    rows_per_w = nrows // NW
    nchunk = rows_per_w // R

    @functools.partial(
        pl.kernel,
        out_type=jax.ShapeDtypeStruct((nrows * VPAD,), jnp.float32),
        mesh=_mesh,
        scratch_types=[
            pltpu.VMEM((R, L), jnp.int32),
            pltpu.VMEM((R, L), jnp.int32),
            pltpu.VMEM((R, L), jnp.int32),
            pltpu.VMEM((R, L), jnp.int32),
            pltpu.VMEM((CV,), jnp.float32),
            pltpu.VMEM((CV,), jnp.float32),
            pltpu.SemaphoreType.DMA,
            pltpu.SemaphoreType.DMA,
            pltpu.SemaphoreType.DMA,
            pltpu.SemaphoreType.DMA,
        ],
        compiler_params=pltpu.CompilerParams(needs_layout_passes=False),
    )
    def sc_counts(x_hbm, out_hbm, idx0, idx1, idx2, idx3, cnt0, cnt1,
                  sem0, sem1, isem0, isem1):
        wid = lax.axis_index("s") * NC + lax.axis_index("c")
        ones = jnp.ones((LN,), jnp.float32)
        zeros = jnp.zeros((LN,), jnp.float32)
        lane = lax.iota(jnp.int32, LN)
        tail_mask = lane >= 8
        idx_b = (idx0, idx1, idx2, idx3)
        cnt_b = (cnt0, cnt1)
        sem_b = (sem0, sem1)
        isem_b = (isem0, isem1)

        ZU = 8

        def zbody(i, carry):
            for u in range(ZU):
                cnt0[pl.ds((i * ZU + u) * LN, LN)] = zeros
                cnt1[pl.ds((i * ZU + u) * LN, LN)] = zeros
            return carry

        lax.fori_loop(0, CV // (LN * ZU), zbody, 0)

        bias_idx0 = lane * VPAD + VOCAB
        bias_idx1 = bias_idx0 + LN * VPAD
        for cbuf in (cnt0, cnt1):
            plsc.store_scatter(cbuf, [bias_idx0], ones)
            plsc.store_scatter(cbuf, [bias_idx1], ones)

        RPI = 2

        def rows_pass(islot, cslot, r0, value_vec, add):
            xs, masks = [], []
            for rr in range(RPI):
                r = r0 + rr
                roff = r * VPAD
                for j in range(13):
                    if j < 12:
                        xs.append(idx_b[islot][r, pl.ds(j * LN, LN)] + roff)
                        masks.append(None)
                    else:
                        xs.append(idx_b[islot][r, pl.ds(L - LN, LN)] + roff)
                        masks.append(tail_mask)
            for fidx, mask in zip(xs, masks):
                if add:
                    plsc.addupdate_scatter(cnt_b[cslot], [fidx], value_vec,
                                           mask=mask)
                else:
                    plsc.store_scatter(cnt_b[cslot], [fidx], zeros, mask=mask)

        def add_pass(islot, cslot):
            def body(i, carry):
                rows_pass(islot, cslot, i * RPI, ones, True)
                return carry
            lax.fori_loop(0, R // RPI, body, 0)

        def zero_pass(islot, cslot):
            def body(i, carry):
                rows_pass(islot, cslot, i * RPI, zeros, False)
                return carry
            lax.fori_loop(0, R // RPI, body, 0)

        def out_copy(cslot, g):
            row0 = wid * rows_per_w + g * R
            return pltpu.make_async_copy(
                cnt_b[cslot], out_hbm.at[pl.ds(row0 * VPAD, CV)], sem_b[cslot])

        def in_copy(islot, g):
            row0 = wid * rows_per_w + g * R
            return pltpu.make_async_copy(
                x_hbm.at[pl.ds(row_off + row0, R)], idx_b[islot],
                isem_b[islot % 2])

        in_copy(0, 0).start()
        in_copy(1, 1).start()
        for g0 in (0, 1):
            in_copy(g0, g0).wait()
            add_pass(g0, g0)
            out_copy(g0, g0).start()
            in_copy(g0 + 2, g0 + 2).start()

        def chunk_body(g, carry):
            def for_islot(islot):
                cslot = islot % 2

                @pl.when(lax.rem(g, 4) == islot)
                def _():
                    out_copy(cslot, g - 2).wait()
                    zero_pass((islot + 2) % 4, cslot)
                    in_copy(islot, g).wait()
                    add_pass(islot, cslot)
                    out_copy(cslot, g).start()

                    @pl.when(g + 2 < nchunk)
                    def _():
                        in_copy((islot + 2) % 4, g + 2).start()
            for islot in range(4):
                for_islot(islot)
            return carry

        lax.fori_loop(2, nchunk, chunk_body, 0)
        out_copy(0, nchunk - 2).wait()
        out_copy(1, nchunk - 1).wait()

    return sc_counts


_sc_counts_a = _make_sc_counts(BH, 0)
_sc_counts_b = _make_sc_counts(BH, BH)


def _prep_body(emb_ref, win_ref, bin_ref, t_ref):
    t_ref[pl.ds(0, VOCAB), :] = lax.dot_general(
        emb_ref[...], win_ref[...], (((1,), (1,)), ((), ())),
        preferred_element_type=jnp.float32)
    t_ref[pl.ds(VOCAB, VPAD - VOCAB), :] = jnp.zeros(
        (VPAD - VOCAB, VEC), jnp.float32)
    t_ref[pl.ds(VOCAB, 1), :] = bin_ref[...]


BM = 1024
NBH = BH // BM


def _compute_block(cnt_ref, t_ref, wout_ref, bout_ref, out_ref):
    h = jnp.dot(cnt_ref[:, 0, :], t_ref[pl.ds(0, 128), :],
                preferred_element_type=jnp.float32)
    for j in range(1, 8):
        h = h + jnp.dot(cnt_ref[:, j, :], t_ref[pl.ds(j * 128, 128), :],
                        preferred_element_type=jnp.float32)
    e0 = (lax.broadcasted_iota(jnp.int32, (BM, VEC), 1) == 0).astype(jnp.float32)
    out_ref[...] = (
        lax.dot_general(wout_ref[...], h, (((1,), (1,)), ((), ())),
                        preferred_element_type=jnp.float32)
        + lax.dot_general(bout_ref[...], e0, (((1,), (1,)), ((), ())),
                          preferred_element_type=jnp.float32))


def _main_a_body(cnt_ref, t_ref, wout_ref, bout_ref, out_ref):
    _compute_block(cnt_ref, t_ref, wout_ref, bout_ref, out_ref)


def _main_b_body(alias_ref, cnt_ref, t_ref, wout_ref, bout_ref, out_ref):
    del alias_ref
    _compute_block(cnt_ref, t_ref, wout_ref, bout_ref, out_ref)


_weight_specs = [
    pl.BlockSpec((VPAD, VEC), lambda i: (0, 0)),
    pl.BlockSpec((VOCAB, VEC), lambda i: (0, 0)),
    pl.BlockSpec((VOCAB, VEC), lambda i: (0, 0)),
]


def kernel(x, embeddings, W_in, b_in, W_out, b_out):
    counts_a = _sc_counts_a(x)
    counts_b = _sc_counts_b(x)
    t = pl.pallas_call(
        _prep_body,
        out_shape=jax.ShapeDtypeStruct((VPAD, VEC), jnp.float32),
    )(embeddings, W_in, b_in.reshape(1, VEC))
    bout_col = jnp.pad(b_out.reshape(VOCAB, 1), ((0, 0), (0, VEC - 1)))
    out_shape = jax.ShapeDtypeStruct((VOCAB, B), jnp.float32)
    out_a = pl.pallas_call(
        _main_a_body,
        grid=(NBH,),
        in_specs=[pl.BlockSpec((BM, 8, 128), lambda i: (i, 0, 0))]
        + _weight_specs,
        out_specs=pl.BlockSpec((VOCAB, BM), lambda i: (0, i)),
        out_shape=out_shape,
    )(counts_a.reshape(BH, 8, 128), t, W_out, bout_col)
    out_t = pl.pallas_call(
        _main_b_body,
        grid=(NBH,),
        in_specs=[pl.BlockSpec(memory_space=pltpu.MemorySpace.HBM)]
        + [pl.BlockSpec((BM, 8, 128), lambda i: (i, 0, 0))]
        + _weight_specs,
        out_specs=pl.BlockSpec((VOCAB, BM), lambda i: (0, i + NBH)),
        out_shape=out_shape,
        input_output_aliases={0: 0},
    )(out_a, counts_b.reshape(BH, 8, 128), t, W_out, bout_col)
    return out_t.T
